# Initial kernel scaffold; baseline (speedup 1.0000x reference)
import jax
import jax.numpy as jnp
from jax.experimental import pallas as pl


def kernel(feature_maps, landmarks):
    raise NotImplementedError("write your pallas kernel here")


# speedup vs baseline: 7457.7580x; 7457.7580x over previous
import functools

import jax
import jax.numpy as jnp
from jax.experimental import pallas as pl
from jax.experimental.pallas import tpu as pltpu

H = 384
W = 384
RAD2 = 40 * 40


def _body(lx_ref, ly_ref, fm_ref, out_ref, acc_ref, *, B, P):
    b = pl.program_id(0)
    p = pl.program_id(1)
    x = lx_ref[b, p]
    y = ly_ref[b, p]

    ph = fm_ref[0, 0, 0]
    pox = fm_ref[0, 1, 0]
    poy = fm_ref[0, 2, 0]

    row = jax.lax.broadcasted_iota(jnp.int32, (H, W), 0)
    col = jax.lax.broadcasted_iota(jnp.int32, (H, W), 1)
    dy = row - y
    dx = col - x
    t = ((dy * dy + dx * dx) <= RAD2).astype(jnp.float32)

    bce = jnp.maximum(ph, 0.0) - ph * t + jnp.log1p(jnp.exp(-jnp.abs(ph)))
    offx = (-dy).astype(jnp.float32) / 40.0
    offy = (-dx).astype(jnp.float32) / 40.0

    bce_sum = jnp.sum(bce)
    cnt = jnp.sum(t)
    sx = jnp.sum(jnp.abs(pox - offx) * t)
    sy = jnp.sum(jnp.abs(poy - offy) * t)

    @pl.when(jnp.logical_and(b == 0, p == 0))
    def _init():
        acc_ref[0] = 0.0
        acc_ref[1] = 0.0
        acc_ref[2] = 0.0
        acc_ref[3] = 0.0

    acc_ref[0] += bce_sum
    acc_ref[1] += cnt
    acc_ref[2] += sx
    acc_ref[3] += sy

    @pl.when(jnp.logical_and(b == B - 1, p == P - 1))
    def _fin():
        total = jnp.float32(B * P * H * W)
        out_ref[0, 0] = (2.0 * acc_ref[0] / total
                         + (acc_ref[2] + acc_ref[3]) / acc_ref[1])


@jax.jit
def kernel(feature_maps, landmarks):
    B, C, height, width = feature_maps.shape
    P = C // 3
    fm = feature_maps.reshape(B, 3, P, height, width)
    lm = landmarks.astype(jnp.int32)
    lx = jnp.clip(lm[..., 0], 1, width - 1)
    ly = jnp.clip(lm[..., 1], 1, height - 1)

    out = pl.pallas_call(
        functools.partial(_body, B=B, P=P),
        grid=(B, P),
        in_specs=[
            pl.BlockSpec(memory_space=pltpu.SMEM),
            pl.BlockSpec(memory_space=pltpu.SMEM),
            pl.BlockSpec((1, 3, 1, height, width),
                         lambda b, p: (b, 0, p, 0, 0)),
        ],
        out_specs=pl.BlockSpec(memory_space=pltpu.SMEM),
        out_shape=jax.ShapeDtypeStruct((1, 1), jnp.float32),
        scratch_shapes=[pltpu.SMEM((4,), jnp.float32)],
    )(lx, ly, fm)
    return out[0, 0]
